# Initial kernel scaffold; baseline (speedup 1.0000x reference)
import jax
import jax.numpy as jnp
from jax.experimental import pallas as pl


def kernel(x, edge_index, edge_type, num_nodes, W, W0):
    raise NotImplementedError("write your pallas kernel here")


# speedup vs baseline: 12.7257x; 12.7257x over previous
import jax
import jax.numpy as jnp
from jax import lax
from jax.experimental import pallas as pl
from jax.experimental.pallas import tpu as pltpu
from jax.experimental.pallas import tpu_sc as plsc

N = 10000
E = 320000
D = 128
R = 8
DH = D // 2

NC = 2
NS = 16
EPT = E // NS
CH = 80
NCHUNK = EPT // CH
NP = 10240
RPT = NP // NS
ZR = 128


def _mm_body(x_ref, w_ref, o_ref):
    o_ref[...] = jnp.dot(x_ref[...], w_ref[0, 0], preferred_element_type=jnp.float32)


def _relation_matmuls(x, Wsplit):
    blk = 2000
    nb = N // blk
    return pl.pallas_call(
        _mm_body,
        grid=(NC, R, nb),
        in_specs=[
            pl.BlockSpec((blk, D), lambda c, r, i: (i, 0)),
            pl.BlockSpec((1, 1, D, DH), lambda c, r, i: (c, r, 0, 0)),
        ],
        out_specs=pl.BlockSpec((blk, DH), lambda c, r, i: (c * R * nb + r * nb + i, 0)),
        out_shape=jax.ShapeDtypeStruct((NC * R * N, DH), jnp.float32),
    )(x, Wsplit)


def _sc_body(xw_hbm, src_hbm, dst_hbm, typ_hbm, part_hbm, degp_hbm,
             acc_sh, deg_sh, zbuf, zdbuf, src_v, dst_v, typ_v, g_v,
             rows_v, ones_v, sem):
    c = lax.axis_index("c")
    s = lax.axis_index("s")
    ebase = s * EPT
    rbase = s * RPT
    goff = c * (R * N)

    zero16 = jnp.zeros((16,), jnp.float32)
    one16 = jnp.ones((16,), jnp.float32)

    def _z(i, _):
        zbuf[i, pl.ds(0, 16)] = zero16
        zbuf[i, pl.ds(16, 16)] = zero16
        zbuf[i, pl.ds(32, 16)] = zero16
        zbuf[i, pl.ds(48, 16)] = zero16
        return 0
    lax.fori_loop(0, ZR, _z, 0)

    for k in range(RPT // ZR):
        pltpu.sync_copy(zbuf, acc_sh.at[pl.ds(rbase + k * ZR, ZR)])

    @pl.when(c == 0)
    def _():
        def _zd(i, _):
            zdbuf[i, :] = zero16
            return 0
        lax.fori_loop(0, RPT, _zd, 0)

        def _o(i, _):
            ones_v[i, :] = one16
            return 0
        lax.fori_loop(0, CH, _o, 0)
        pltpu.sync_copy(zdbuf, deg_sh.at[pl.ds(rbase, RPT)])

    plsc.subcore_barrier()

    def _chunk(it, _):
        base = ebase + it * CH
        pltpu.sync_copy(src_hbm.at[pl.ds(base, CH)], src_v)
        pltpu.sync_copy(dst_hbm.at[pl.ds(base, CH)], dst_v)
        pltpu.sync_copy(typ_hbm.at[pl.ds(base, CH)], typ_v)

        def _gidx(i, _):
            sl = pl.ds(i * 16, 16)
            g_v[sl] = goff + typ_v[sl] * N + src_v[sl]
            return 0
        lax.fori_loop(0, CH // 16, _gidx, 0)

        pltpu.async_copy(xw_hbm.at[g_v], rows_v, sem).wait()
        pltpu.sync_copy(rows_v, acc_sh.at[dst_v], add=True)

        @pl.when(c == 0)
        def _():
            pltpu.sync_copy(ones_v, deg_sh.at[dst_v], add=True)
        return 0
    lax.fori_loop(0, NCHUNK, _chunk, 0)

    plsc.subcore_barrier()

    for k in range(RPT // ZR):
        pltpu.sync_copy(acc_sh.at[pl.ds(rbase + k * ZR, ZR)], zbuf)
        pltpu.sync_copy(zbuf, part_hbm.at[c, pl.ds(rbase + k * ZR, ZR)])

    @pl.when(c == 0)
    def _():
        pltpu.sync_copy(deg_sh.at[pl.ds(rbase, RPT)], zdbuf)
        pltpu.sync_copy(zdbuf, degp_hbm.at[pl.ds(rbase, RPT)])


def _sc_aggregate(xw, src, dst, typ):
    mesh = plsc.VectorSubcoreMesh(core_axis_name="c", subcore_axis_name="s")
    f = pl.kernel(
        _sc_body,
        out_type=(
            jax.ShapeDtypeStruct((NC, NP, DH), jnp.float32),
            jax.ShapeDtypeStruct((NP, 16), jnp.float32),
        ),
        mesh=mesh,
        compiler_params=pltpu.CompilerParams(use_tc_tiling_on_sc=False),
        scratch_types=[
            pltpu.VMEM_SHARED((NP, DH), jnp.float32),
            pltpu.VMEM_SHARED((NP, 16), jnp.float32),
            pltpu.VMEM((ZR, DH), jnp.float32),
            pltpu.VMEM((RPT, 16), jnp.float32),
            pltpu.VMEM((CH,), jnp.int32),
            pltpu.VMEM((CH,), jnp.int32),
            pltpu.VMEM((CH,), jnp.int32),
            pltpu.VMEM((CH,), jnp.int32),
            pltpu.VMEM((CH, DH), jnp.float32),
            pltpu.VMEM((CH, 16), jnp.float32),
            pltpu.SemaphoreType.DMA,
        ],
    )
    return f(xw, src, dst, typ)


def _final_body(p_ref, deg_ref, x_ref, w0_ref, o_ref):
    p = p_ref[...]
    o = jnp.concatenate([p[0], p[1]], axis=-1) + jnp.dot(
        x_ref[...], w0_ref[...], preferred_element_type=jnp.float32)
    d = jnp.maximum(deg_ref[:, 0:1], 1.0)
    o_ref[...] = o / d


def _finalize(part, degp, x, W0):
    blk = 2000
    nb = N // blk
    return pl.pallas_call(
        _final_body,
        grid=(nb,),
        in_specs=[
            pl.BlockSpec((NC, blk, DH), lambda i: (0, i, 0)),
            pl.BlockSpec((blk, 16), lambda i: (i, 0)),
            pl.BlockSpec((blk, D), lambda i: (i, 0)),
            pl.BlockSpec((D, D), lambda i: (0, 0)),
        ],
        out_specs=pl.BlockSpec((blk, D), lambda i: (i, 0)),
        out_shape=jax.ShapeDtypeStruct((N, D), jnp.float32),
    )(part, degp, x, W0)


@jax.jit
def _run(x, edge_index, edge_type, W, W0):
    Wsplit = W.reshape(R, D, NC, DH).transpose(2, 0, 1, 3)
    xw = _relation_matmuls(x, Wsplit)
    src = edge_index[0]
    dst = edge_index[1]
    part, degp = _sc_aggregate(xw, src, dst, edge_type)
    return _finalize(part, degp, x, W0)


def kernel(x, edge_index, edge_type, num_nodes, W, W0):
    return _run(x, edge_index, edge_type, W, W0)


# speedup vs baseline: 32.0954x; 2.5221x over previous
import jax
import jax.numpy as jnp
from jax import lax
from jax.experimental import pallas as pl
from jax.experimental.pallas import tpu as pltpu
from jax.experimental.pallas import tpu_sc as plsc

N = 10000
E = 320000
D = 128
R = 8
DH = D // 2

NC = 2
NS = 16
CH = 80
ROWS = E // CH
RPTILE = ROWS // NS
EPB = 50
NBLK = RPTILE // EPB
NSLOT = 5
DEG_PB = EPB // 2
NP = 10240
RPT = NP // NS
ZR = 128


def _mm_body(x_ref, w_ref, o_ref):
    o_ref[...] = jnp.dot(x_ref[...], w_ref[0, 0], preferred_element_type=jnp.float32)


def _relation_matmuls(x, Wsplit):
    blk = 2000
    nb = N // blk
    return pl.pallas_call(
        _mm_body,
        grid=(NC, R, nb),
        in_specs=[
            pl.BlockSpec((blk, D), lambda c, r, i: (i, 0)),
            pl.BlockSpec((1, 1, D, DH), lambda c, r, i: (c, r, 0, 0)),
        ],
        out_specs=pl.BlockSpec((blk, DH), lambda c, r, i: (c * R * nb + r * nb + i, 0)),
        out_shape=jax.ShapeDtypeStruct((NC * R * N, DH), jnp.float32),
    )(x, Wsplit)


def _sc_body(xw_hbm, src_hbm, dst_hbm, typ_hbm, part_hbm, degp_hbm,
             acc_sh, deg_sh, zbuf, zdbuf, src2_v, dst2_v, typ2_v, g2_v,
             rows_v, ones_v,
             sg0, sg1, sg2, sg3, sg4, ss0, ss1, ss2, ss3, ss4, sem_d):
    sem_g = (sg0, sg1, sg2, sg3, sg4)
    sem_s = (ss0, ss1, ss2, ss3, ss4)
    c = lax.axis_index("c")
    s = lax.axis_index("s")
    rbase = s * RPT
    goff = c * (R * N)

    zero16 = jnp.zeros((16,), jnp.float32)
    one16 = jnp.ones((16,), jnp.float32)

    def _z(i, _):
        zbuf[i, pl.ds(0, 16)] = zero16
        zbuf[i, pl.ds(16, 16)] = zero16
        zbuf[i, pl.ds(32, 16)] = zero16
        zbuf[i, pl.ds(48, 16)] = zero16
        return 0
    lax.fori_loop(0, ZR, _z, 0)

    def _zd(i, _):
        zdbuf[i, :] = zero16
        return 0
    lax.fori_loop(0, RPT, _zd, 0)

    def _o(i, _):
        ones_v[i, :] = one16
        return 0
    lax.fori_loop(0, CH, _o, 0)

    for k in range(RPT // ZR):
        pltpu.sync_copy(zbuf, acc_sh.at[pl.ds(rbase + k * ZR, ZR)])
    pltpu.sync_copy(zdbuf, deg_sh.at[pl.ds(rbase, RPT)])

    plsc.subcore_barrier()

    def _fire_g(j, b):
        pltpu.async_copy(xw_hbm.at[g2_v.at[j]], rows_v.at[b], sem_g[b])

    def _wait_g(b):
        pltpu.make_async_copy(xw_hbm.at[g2_v.at[0]], rows_v.at[b],
                              sem_g[b]).wait()

    def _wait_s(b):
        pltpu.make_async_copy(rows_v.at[b], acc_sh.at[dst2_v.at[0]],
                              sem_s[b]).wait()

    def _wait_d():
        pltpu.make_async_copy(ones_v, deg_sh.at[dst2_v.at[0]], sem_d).wait()

    def _block(k, _):
        rowb = s * RPTILE + k * EPB
        pltpu.sync_copy(src_hbm.at[pl.ds(rowb, EPB)], src2_v)
        pltpu.sync_copy(dst_hbm.at[pl.ds(rowb, EPB)], dst2_v)
        pltpu.sync_copy(typ_hbm.at[pl.ds(rowb, EPB)], typ2_v)

        def _gidx(r, _):
            for i in range(CH // 16):
                sl = pl.ds(i * 16, 16)
                g2_v[r, sl] = goff + typ2_v[r, sl] * N + src2_v[r, sl]
            return 0
        lax.fori_loop(0, EPB, _gidx, 0)

        for b in range(3):
            _fire_g(b, b)

        def _step(gg, _):
            for b in range(NSLOT):
                j = gg * NSLOT + b
                _wait_g(b)
                pltpu.async_copy(rows_v.at[b], acc_sh.at[dst2_v.at[j]],
                                 sem_s[b], add=True)

                @pl.when((j % 2) == c)
                def _():
                    pltpu.async_copy(ones_v, deg_sh.at[dst2_v.at[j]], sem_d,
                                     add=True)

                @pl.when(j >= 2)
                def _():
                    _wait_s((b + 3) % NSLOT)

                @pl.when(j + 3 < EPB)
                def _():
                    _fire_g(j + 3, (b + 3) % NSLOT)
            return 0
        lax.fori_loop(0, EPB // NSLOT, _step, 0)

        _wait_s((EPB - 2) % NSLOT)
        _wait_s((EPB - 1) % NSLOT)

        def _dd(i, _):
            _wait_d()
            return 0
        lax.fori_loop(0, DEG_PB, _dd, 0)
        return 0
    lax.fori_loop(0, NBLK, _block, 0)

    plsc.subcore_barrier()

    for k in range(RPT // ZR):
        pltpu.sync_copy(acc_sh.at[pl.ds(rbase + k * ZR, ZR)], zbuf)
        pltpu.sync_copy(zbuf, part_hbm.at[c, pl.ds(rbase + k * ZR, ZR)])
    pltpu.sync_copy(deg_sh.at[pl.ds(rbase, RPT)], zdbuf)
    pltpu.sync_copy(zdbuf, degp_hbm.at[c, pl.ds(rbase, RPT)])


def _sc_aggregate(xw, src2, dst2, typ2):
    mesh = plsc.VectorSubcoreMesh(core_axis_name="c", subcore_axis_name="s")
    f = pl.kernel(
        _sc_body,
        out_type=(
            jax.ShapeDtypeStruct((NC, NP, DH), jnp.float32),
            jax.ShapeDtypeStruct((NC, NP, 16), jnp.float32),
        ),
        mesh=mesh,
        compiler_params=pltpu.CompilerParams(use_tc_tiling_on_sc=False),
        scratch_types=[
            pltpu.VMEM_SHARED((NP, DH), jnp.float32),
            pltpu.VMEM_SHARED((NP, 16), jnp.float32),
            pltpu.VMEM((ZR, DH), jnp.float32),
            pltpu.VMEM((RPT, 16), jnp.float32),
            pltpu.VMEM((EPB, CH), jnp.int32),
            pltpu.VMEM((EPB, CH), jnp.int32),
            pltpu.VMEM((EPB, CH), jnp.int32),
            pltpu.VMEM((EPB, CH), jnp.int32),
            pltpu.VMEM((NSLOT, CH, DH), jnp.float32),
            pltpu.VMEM((CH, 16), jnp.float32),
            pltpu.SemaphoreType.DMA,
            pltpu.SemaphoreType.DMA,
            pltpu.SemaphoreType.DMA,
            pltpu.SemaphoreType.DMA,
            pltpu.SemaphoreType.DMA,
            pltpu.SemaphoreType.DMA,
            pltpu.SemaphoreType.DMA,
            pltpu.SemaphoreType.DMA,
            pltpu.SemaphoreType.DMA,
            pltpu.SemaphoreType.DMA,
            pltpu.SemaphoreType.DMA,
        ],
    )
    return f(xw, src2, dst2, typ2)


def _final_body(p_ref, deg_ref, x_ref, w0_ref, o_ref):
    p = p_ref[...]
    o = jnp.concatenate([p[0], p[1]], axis=-1) + jnp.dot(
        x_ref[...], w0_ref[...], preferred_element_type=jnp.float32)
    dg = deg_ref[...]
    d = jnp.maximum(dg[0, :, 0:1] + dg[1, :, 0:1], 1.0)
    o_ref[...] = o / d


def _finalize(part, degp, x, W0):
    blk = 2000
    nb = N // blk
    return pl.pallas_call(
        _final_body,
        grid=(nb,),
        in_specs=[
            pl.BlockSpec((NC, blk, DH), lambda i: (0, i, 0)),
            pl.BlockSpec((NC, blk, 16), lambda i: (0, i, 0)),
            pl.BlockSpec((blk, D), lambda i: (i, 0)),
            pl.BlockSpec((D, D), lambda i: (0, 0)),
        ],
        out_specs=pl.BlockSpec((blk, D), lambda i: (i, 0)),
        out_shape=jax.ShapeDtypeStruct((N, D), jnp.float32),
    )(part, degp, x, W0)


@jax.jit
def _run(x, edge_index, edge_type, W, W0):
    Wsplit = W.reshape(R, D, NC, DH).transpose(2, 0, 1, 3)
    xw = _relation_matmuls(x, Wsplit)
    src2 = edge_index[0].reshape(ROWS, CH)
    dst2 = edge_index[1].reshape(ROWS, CH)
    typ2 = edge_type.reshape(ROWS, CH)
    part, degp = _sc_aggregate(xw, src2, dst2, typ2)
    return _finalize(part, degp, x, W0)


def kernel(x, edge_index, edge_type, num_nodes, W, W0):
    return _run(x, edge_index, edge_type, W, W0)


# speedup vs baseline: 36.2085x; 1.1282x over previous
import jax
import jax.numpy as jnp
from jax import lax
from jax.experimental import pallas as pl
from jax.experimental.pallas import tpu as pltpu
from jax.experimental.pallas import tpu_sc as plsc

N = 10000
E = 320000
D = 128
R = 8
R1 = R + 1
DH = D // 2

NC = 2
NS = 16
CH = 80
ROWS = E // CH
RPTILE = ROWS // NS
EPB = 50
NBLK = RPTILE // EPB
NSLOT = 5
NP = 10240
RPT = NP // NS
ZR = 128


def _mm_body(x_ref, w_ref, o_ref):
    x = x_ref[...]
    for r in range(R1):
        res = jnp.dot(x, w_ref[r], preferred_element_type=jnp.float32)
        o_ref[0, r] = res[:, :DH]
        o_ref[1, r] = res[:, DH:]


def _relation_matmuls(xp, Wf):
    blk = 2048
    nb = NP // blk
    return pl.pallas_call(
        _mm_body,
        grid=(nb,),
        in_specs=[
            pl.BlockSpec((blk, D), lambda i: (i, 0)),
            pl.BlockSpec((R1, D, D), lambda i: (0, 0, 0)),
        ],
        out_specs=pl.BlockSpec((NC, R1, blk, DH), lambda i: (0, 0, i, 0)),
        out_shape=jax.ShapeDtypeStruct((NC, R1, NP, DH), jnp.float32),
    )(xp, Wf)


def _sc_body(xw_hbm, src_hbm, dst_hbm, typ_hbm, out_hbm,
             acc_sh, deg_sh, zbuf, zdbuf, src2_v, dst2_v, typ2_v, g2_v,
             rows_v, ones_v,
             sg0, sg1, sg2, sg3, sg4, ss0, ss1, ss2, ss3, ss4, sem_d):
    sem_g = (sg0, sg1, sg2, sg3, sg4)
    sem_s = (ss0, ss1, ss2, ss3, ss4)
    c = lax.axis_index("c")
    s = lax.axis_index("s")
    rbase = s * RPT
    goff = c * (R1 * NP)
    w0off = goff + R * NP

    zero16 = jnp.zeros((16,), jnp.float32)
    one16 = jnp.ones((16,), jnp.float32)

    for k in range(RPT // ZR):
        off = rbase + k * ZR
        pltpu.sync_copy(xw_hbm.at[pl.ds(w0off + off, ZR)], zbuf)
        pltpu.sync_copy(zbuf, acc_sh.at[pl.ds(off, ZR)])

    def _zd(i, _):
        zdbuf[i, :] = zero16
        return 0
    lax.fori_loop(0, RPT, _zd, 0)

    def _o(i, _):
        ones_v[i, :] = one16
        return 0
    lax.fori_loop(0, CH, _o, 0)

    pltpu.sync_copy(zdbuf, deg_sh.at[pl.ds(rbase, RPT)])

    plsc.subcore_barrier()

    def _fire_g(j, b):
        pltpu.async_copy(xw_hbm.at[g2_v.at[j]], rows_v.at[b], sem_g[b])

    def _wait_g(b):
        pltpu.make_async_copy(xw_hbm.at[g2_v.at[0]], rows_v.at[b],
                              sem_g[b]).wait()

    def _wait_s(b):
        pltpu.make_async_copy(rows_v.at[b], acc_sh.at[dst2_v.at[0]],
                              sem_s[b]).wait()

    def _wait_d():
        pltpu.make_async_copy(ones_v, deg_sh.at[dst2_v.at[0]], sem_d).wait()

    def _block(k, _):
        rowb = s * RPTILE + k * EPB
        pltpu.sync_copy(src_hbm.at[pl.ds(rowb, EPB)], src2_v)
        pltpu.sync_copy(dst_hbm.at[pl.ds(rowb, EPB)], dst2_v)
        pltpu.sync_copy(typ_hbm.at[pl.ds(rowb, EPB)], typ2_v)

        def _gidx(r, _):
            for i in range(CH // 16):
                sl = pl.ds(i * 16, 16)
                g2_v[r, sl] = goff + typ2_v[r, sl] * NP + src2_v[r, sl]
            return 0
        lax.fori_loop(0, EPB, _gidx, 0)

        for b in range(3):
            _fire_g(b, b)

        def _step(gg, _):
            for b in range(NSLOT):
                j = gg * NSLOT + b
                _wait_g(b)
                pltpu.async_copy(rows_v.at[b], acc_sh.at[dst2_v.at[j]],
                                 sem_s[b], add=True)
                pltpu.async_copy(ones_v, deg_sh.at[dst2_v.at[j]], sem_d,
                                 add=True)

                @pl.when(j >= 2)
                def _():
                    _wait_s((b + 3) % NSLOT)

                @pl.when(j + 3 < EPB)
                def _():
                    _fire_g(j + 3, (b + 3) % NSLOT)
            return 0
        lax.fori_loop(0, EPB // NSLOT, _step, 0)

        _wait_s((EPB - 2) % NSLOT)
        _wait_s((EPB - 1) % NSLOT)

        def _dd(i, _):
            _wait_d()
            return 0
        lax.fori_loop(0, EPB, _dd, 0)
        return 0
    lax.fori_loop(0, NBLK, _block, 0)

    plsc.subcore_barrier()

    pltpu.sync_copy(deg_sh.at[pl.ds(rbase, RPT)], zdbuf)
    for k in range(RPT // ZR):
        pltpu.sync_copy(acc_sh.at[pl.ds(rbase + k * ZR, ZR)], zbuf)

        def _div(r, _):
            dvec = jnp.maximum(zdbuf[k * ZR + r, :], 1.0)
            for i in range(DH // 16):
                sl = pl.ds(i * 16, 16)
                zbuf[r, sl] = zbuf[r, sl] / dvec
            return 0
        lax.fori_loop(0, ZR, _div, 0)
        pltpu.sync_copy(zbuf, out_hbm.at[c, pl.ds(rbase + k * ZR, ZR)])


def _sc_aggregate(xw, src2, dst2, typ2):
    mesh = plsc.VectorSubcoreMesh(core_axis_name="c", subcore_axis_name="s")
    f = pl.kernel(
        _sc_body,
        out_type=jax.ShapeDtypeStruct((NC, NP, DH), jnp.float32),
        mesh=mesh,
        compiler_params=pltpu.CompilerParams(use_tc_tiling_on_sc=False),
        scratch_types=[
            pltpu.VMEM_SHARED((NP, DH), jnp.float32),
            pltpu.VMEM_SHARED((NP, 16), jnp.float32),
            pltpu.VMEM((ZR, DH), jnp.float32),
            pltpu.VMEM((RPT, 16), jnp.float32),
            pltpu.VMEM((EPB, CH), jnp.int32),
            pltpu.VMEM((EPB, CH), jnp.int32),
            pltpu.VMEM((EPB, CH), jnp.int32),
            pltpu.VMEM((EPB, CH), jnp.int32),
            pltpu.VMEM((NSLOT, CH, DH), jnp.float32),
            pltpu.VMEM((CH, 16), jnp.float32),
            pltpu.SemaphoreType.DMA,
            pltpu.SemaphoreType.DMA,
            pltpu.SemaphoreType.DMA,
            pltpu.SemaphoreType.DMA,
            pltpu.SemaphoreType.DMA,
            pltpu.SemaphoreType.DMA,
            pltpu.SemaphoreType.DMA,
            pltpu.SemaphoreType.DMA,
            pltpu.SemaphoreType.DMA,
            pltpu.SemaphoreType.DMA,
            pltpu.SemaphoreType.DMA,
        ],
    )
    return f(xw, src2, dst2, typ2)


@jax.jit
def _run(x, edge_index, edge_type, W, W0):
    Wf = jnp.concatenate([W, W0[None]], axis=0)
    xp = jnp.pad(x, ((0, NP - N), (0, 0)))
    xw = _relation_matmuls(xp, Wf).reshape(NC * R1 * NP, DH)
    src2 = edge_index[0].reshape(ROWS, CH)
    dst2 = edge_index[1].reshape(ROWS, CH)
    typ2 = edge_type.reshape(ROWS, CH)
    halves = _sc_aggregate(xw, src2, dst2, typ2)
    return jnp.concatenate([halves[0, :N], halves[1, :N]], axis=1)


def kernel(x, edge_index, edge_type, num_nodes, W, W0):
    return _run(x, edge_index, edge_type, W, W0)
